# baseline (speedup 1.0000x reference)
import jax
import jax.numpy as jnp
from jax.experimental import pallas as pl
from jax.experimental.pallas import tpu as pltpu

H = 128
W = 128
TILE = 16
N = 1024
NTX = W // TILE
NTY = H // TILE
NT = NTX * NTY
K2 = 256
NCH = N // K2
P = TILE * TILE
BKGD = 1.0


def _prep_kernel(attrs_ref, covu_ref, prep_ref, rects_ref, radii_ref):
    ca = covu_ref[0:1, :]
    cb = covu_ref[1:2, :]
    cd = covu_ref[2:3, :]
    det = ca * cd - cb * cb
    mid = 0.5 * (ca + cd)
    root = jnp.sqrt(jnp.maximum(mid * mid - det, 0.1))
    radii_ref[0:1, :] = jnp.ceil(3.0 * jnp.sqrt(mid + root))

    mx = attrs_ref[0:1, :]
    my = attrs_ref[1:2, :]
    ca = attrs_ref[2:3, :]
    cb = attrs_ref[3:4, :]
    cd = attrs_ref[4:5, :]
    det = ca * cd - cb * cb
    prep_ref[0:1, :] = mx
    prep_ref[1:2, :] = my
    prep_ref[2:3, :] = cd / det
    prep_ref[3:4, :] = ca / det
    prep_ref[4:5, :] = -cb / det
    prep_ref[5:16, :] = attrs_ref[5:16, :]

    mid = 0.5 * (ca + cd)
    root = jnp.sqrt(jnp.maximum(mid * mid - det, 0.1))
    rad = jnp.ceil(3.0 * jnp.sqrt(mid + root))
    rects_ref[0:1, :] = jnp.clip(mx - rad, 0.0, W - 1.0)
    rects_ref[1:2, :] = jnp.clip(mx + rad, 0.0, W - 1.0)
    rects_ref[2:3, :] = jnp.clip(my - rad, 0.0, H - 1.0)
    rects_ref[3:4, :] = jnp.clip(my + rad, 0.0, H - 1.0)
    rects_ref[4:8, :] = jnp.zeros((4, N), jnp.float32)


def _prep(attrs, covu):
    return pl.pallas_call(
        _prep_kernel,
        out_shape=[
            jax.ShapeDtypeStruct((16, N), jnp.float32),
            jax.ShapeDtypeStruct((8, N), jnp.float32),
            jax.ShapeDtypeStruct((1, N), jnp.float32),
        ],
    )(attrs, covu)


def _raster_kernel(cnt_ref, packed_ref, out_ref):
    t = pl.program_id(0)
    cnt = cnt_ref[t]
    nch = (cnt + K2 - 1) // K2
    w0 = (t % NTX) * TILE
    h0 = (t // NTX) * TILE

    pp = jax.lax.broadcasted_iota(jnp.int32, (1, P), 1)
    pxf = (w0 + pp % TILE).astype(jnp.float32)
    pyf = (h0 + pp // TILE).astype(jnp.float32)

    tril = (jax.lax.broadcasted_iota(jnp.int32, (K2, K2), 1) <
            jax.lax.broadcasted_iota(jnp.int32, (K2, K2), 0)).astype(jnp.float32)
    gidx = jax.lax.broadcasted_iota(jnp.int32, (K2, 1), 0)

    def chunk_body(c, carry):
        t_carry, acc_r, acc_g, acc_b, acc_a = carry
        blk = packed_ref[0, c]
        mx = blk[:, 0:1]
        my = blk[:, 1:2]
        i00 = blk[:, 2:3]
        i11 = blk[:, 3:4]
        i01 = blk[:, 4:5]
        op = blk[:, 5:6]
        colr = blk[:, 6:7]
        colg = blk[:, 7:8]
        colb = blk[:, 8:9]
        valid = (gidx + c * K2) < cnt
        dx = pxf - mx
        dy = pyf - my
        quad = dx * dx * i00 + dy * dy * i11 + 2.0 * (dx * dy) * i01
        gw = jnp.exp(-0.5 * quad)
        alpha = jnp.where(valid, jnp.minimum(gw * op, 0.99), 0.0)
        lg = jnp.log1p(-alpha)
        sexc = jax.lax.dot_general(
            tril, lg, (((1,), (0,)), ((), ())),
            precision=jax.lax.Precision.HIGHEST,
            preferred_element_type=jnp.float32)
        wgt = t_carry * jnp.exp(sexc) * alpha
        acc_r = acc_r + jnp.sum(wgt * colr, axis=0, keepdims=True)
        acc_g = acc_g + jnp.sum(wgt * colg, axis=0, keepdims=True)
        acc_b = acc_b + jnp.sum(wgt * colb, axis=0, keepdims=True)
        acc_a = acc_a + jnp.sum(wgt, axis=0, keepdims=True)
        t_carry = t_carry * jnp.exp(jnp.sum(lg, axis=0, keepdims=True))
        return t_carry, acc_r, acc_g, acc_b, acc_a

    init = (jnp.ones((1, P), jnp.float32),) + \
           tuple(jnp.zeros((1, P), jnp.float32) for _ in range(4))
    _, acc_r, acc_g, acc_b, acc_a = jax.lax.fori_loop(0, nch, chunk_body, init)

    resid = (1.0 - acc_a) * BKGD
    out_ref[0, 0:1, :] = acc_r + resid
    out_ref[0, 1:2, :] = acc_g + resid
    out_ref[0, 2:3, :] = acc_b + resid
    out_ref[0, 3:4, :] = acc_a


def _raster(cnt, packed):
    grid_spec = pltpu.PrefetchScalarGridSpec(
        num_scalar_prefetch=1,
        grid=(NT,),
        in_specs=[
            pl.BlockSpec((1, NCH, K2, 16), lambda t, cnt_ref: (t, 0, 0, 0)),
        ],
        out_specs=[
            pl.BlockSpec((1, 8, P), lambda t, cnt_ref: (t, 0, 0)),
        ],
    )
    return pl.pallas_call(
        _raster_kernel,
        grid_spec=grid_spec,
        out_shape=[jax.ShapeDtypeStruct((NT, 8, P), jnp.float32)],
    )(cnt, packed)[0]


@jax.jit
def kernel(means2D, cov2d, color, opacity, depths):
    order = jnp.argsort(depths)
    attrs = jnp.stack([
        means2D[:, 0], means2D[:, 1],
        cov2d[:, 0, 0], cov2d[:, 0, 1], cov2d[:, 1, 1],
        opacity[:, 0],
        color[:, 0], color[:, 1], color[:, 2],
    ], axis=0)[:, order]
    attrs = jnp.concatenate(
        [attrs, jnp.zeros((16 - attrs.shape[0], N), jnp.float32)], axis=0)
    covu = jnp.stack([cov2d[:, 0, 0], cov2d[:, 0, 1], cov2d[:, 1, 1]], axis=0)
    covu = jnp.concatenate(
        [covu, jnp.zeros((8 - covu.shape[0], N), jnp.float32)], axis=0)

    prep, rects, rad = _prep(attrs, covu)

    rminx, rmaxx, rminy, rmaxy = rects[0], rects[1], rects[2], rects[3]
    wox = (jnp.arange(NTX, dtype=jnp.float32) * TILE)
    hoy = (jnp.arange(NTY, dtype=jnp.float32) * TILE)
    maskx = (jnp.minimum(rmaxx[None, :], wox[:, None] + (TILE - 1.0)) >
             jnp.maximum(rminx[None, :], wox[:, None]))
    masky = (jnp.minimum(rmaxy[None, :], hoy[:, None] + (TILE - 1.0)) >
             jnp.maximum(rminy[None, :], hoy[:, None]))
    mask = (masky[:, None, :] & maskx[None, :, :]).reshape(NT, N)
    cnt = jnp.sum(mask, axis=-1).astype(jnp.int32)
    idx = jnp.argsort(~mask, axis=-1, stable=True)
    packed = jnp.transpose(prep, (1, 0))[idx]
    packed = jnp.where(
        jnp.arange(N, dtype=jnp.int32)[None, :, None] < cnt[:, None, None],
        packed, 0.0)
    packed = packed.reshape(NT, NCH, K2, 16)

    out = _raster(cnt, packed)
    img = out[:, :4, :].reshape(NTY, NTX, 4, TILE, TILE)
    img = jnp.transpose(img, (0, 3, 1, 4, 2)).reshape(H, W, 4)
    return img[:, :, :3], img[:, :, 3:4], rad[0]


# speedup vs baseline: 2.9103x; 2.9103x over previous
import functools

import jax
import jax.numpy as jnp
from jax.experimental import pallas as pl
from jax.experimental.pallas import tpu as pltpu
from jax.experimental.pallas import tpu_sc as plsc

H = 128
W = 128
TILE = 16
N = 1024
NTX = W // TILE
NTY = H // TILE
NT = NTX * NTY
K2 = 256
NCH = N // K2
P = TILE * TILE
BKGD = 1.0


def _prep_kernel(attrs_ref, covu_ref, prep_ref, rects_ref, radii_ref):
    ca = covu_ref[0:1, :]
    cb = covu_ref[1:2, :]
    cd = covu_ref[2:3, :]
    det = ca * cd - cb * cb
    mid = 0.5 * (ca + cd)
    root = jnp.sqrt(jnp.maximum(mid * mid - det, 0.1))
    radii_ref[0:1, :] = jnp.ceil(3.0 * jnp.sqrt(mid + root))

    mx = attrs_ref[0:1, :]
    my = attrs_ref[1:2, :]
    ca = attrs_ref[2:3, :]
    cb = attrs_ref[3:4, :]
    cd = attrs_ref[4:5, :]
    det = ca * cd - cb * cb
    prep_ref[0:1, :] = mx
    prep_ref[1:2, :] = my
    prep_ref[2:3, :] = cd / det
    prep_ref[3:4, :] = ca / det
    prep_ref[4:5, :] = -cb / det
    prep_ref[5:16, :] = attrs_ref[5:16, :]

    mid = 0.5 * (ca + cd)
    root = jnp.sqrt(jnp.maximum(mid * mid - det, 0.1))
    rad = jnp.ceil(3.0 * jnp.sqrt(mid + root))
    rects_ref[0:1, :] = jnp.clip(mx - rad, 0.0, W - 1.0)
    rects_ref[1:2, :] = jnp.clip(mx + rad, 0.0, W - 1.0)
    rects_ref[2:3, :] = jnp.clip(my - rad, 0.0, H - 1.0)
    rects_ref[3:4, :] = jnp.clip(my + rad, 0.0, H - 1.0)
    rects_ref[4:8, :] = jnp.zeros((4, N), jnp.float32)


def _prep(attrs, covu):
    return pl.pallas_call(
        _prep_kernel,
        out_shape=[
            jax.ShapeDtypeStruct((16, N), jnp.float32),
            jax.ShapeDtypeStruct((8, N), jnp.float32),
            jax.ShapeDtypeStruct((1, N), jnp.float32),
        ],
    )(attrs, covu)


NC = 2
NS = 16
NWORK = NC * NS
L = 16


def _bin_kernel(rects_hbm, prep_hbm, orig_hbm, packed_hbm, counts_hbm,
                rects_v, prep_v, idx_v, pk_v, cnt_v, orig_v, base_v, vals_v):
    wid = jax.lax.axis_index("s") * NC + jax.lax.axis_index("c")
    pltpu.sync_copy(rects_hbm, rects_v)
    pltpu.sync_copy(prep_hbm, prep_v)
    lanes = jax.lax.iota(jnp.int32, L)

    for j in range(NT // NWORK):
        t = wid * (NT // NWORK) + j
        pltpu.sync_copy(orig_hbm.at[t], orig_v)

        def zero_body(i, _):
            idx_v[pl.ds(i * L, L)] = jnp.zeros((L,), jnp.int32)
            return 0
        jax.lax.fori_loop(0, N // L, zero_body, 0)

        base_v[...] = jnp.zeros((L,), jnp.int32)
        vals_v[...] = lanes

        def bin_body(g, _):
            base = base_v[...]
            vals = vals_v[...]
            wxv = orig_v[0, :]
            hyv = orig_v[1, :]
            rminx = rects_v[0, pl.ds(g * L, L)]
            rmaxx = rects_v[1, pl.ds(g * L, L)]
            rminy = rects_v[2, pl.ds(g * L, L)]
            rmaxy = rects_v[3, pl.ds(g * L, L)]
            m1 = (jnp.minimum(rmaxx, wxv + (TILE - 1.0)) >
                  jnp.maximum(rminx, wxv))
            m2 = (jnp.minimum(rmaxy, hyv + (TILE - 1.0)) >
                  jnp.maximum(rminy, hyv))
            ones = jnp.full((L,), 1, jnp.int32)
            zeros = jnp.full((L,), 0, jnp.int32)
            mi = jnp.where(m1, ones, zeros) * jnp.where(m2, ones, zeros)
            m = mi > zeros
            pos = base + plsc.cumsum(mi) - mi
            plsc.store_scatter(idx_v, [pos], vals, mask=m)
            base_v[...] = base + plsc.all_reduce_population_count(m)
            vals_v[...] = vals + L
            return 0
        jax.lax.fori_loop(0, N // L, bin_body, 0)
        base = base_v[...]

        vals_v[...] = lanes

        def gat_body(g2, _):
            gidx = vals_v[...]
            members = idx_v[pl.ds(g2 * L, L)]
            for a in range(9):
                asplat = jnp.full((L,), a, jnp.int32)
                vals = plsc.load_gather(prep_v, [asplat, members])
                plsc.store_scatter(pk_v, [asplat, gidx], vals)
            vals_v[...] = gidx + L
            return 0
        jax.lax.fori_loop(0, N // L, gat_body, 0)

        for c in range(NCH):
            pltpu.sync_copy(pk_v.at[:, pl.ds(c * K2, K2)], packed_hbm.at[t, c])
        cnt_v[...] = base
        pltpu.sync_copy(cnt_v, counts_hbm.at[t])


def _binning(rects, prep, origins):
    mesh = plsc.VectorSubcoreMesh(core_axis_name="c", subcore_axis_name="s")
    run = pl.kernel(
        _bin_kernel, mesh=mesh,
        compiler_params=pltpu.CompilerParams(needs_layout_passes=False),
        out_type=[
            jax.ShapeDtypeStruct((NT, NCH, 16, K2), jnp.float32),
            jax.ShapeDtypeStruct((NT, L), jnp.int32),
        ],
        scratch_types=[
            pltpu.VMEM((8, N), jnp.float32),
            pltpu.VMEM((16, N), jnp.float32),
            pltpu.VMEM((N,), jnp.int32),
            pltpu.VMEM((16, N), jnp.float32),
            pltpu.VMEM((L,), jnp.int32),
            pltpu.VMEM((2, L), jnp.float32),
            pltpu.VMEM((L,), jnp.int32),
            pltpu.VMEM((L,), jnp.int32),
        ],
    )
    return run(rects, prep, origins)


def _raster_kernel(cnt_ref, packed_ref, out_ref):
    t = pl.program_id(0)
    cnt = cnt_ref[t]
    nch = (cnt + K2 - 1) // K2
    w0 = (t % NTX) * TILE
    h0 = (t // NTX) * TILE

    pp = jax.lax.broadcasted_iota(jnp.int32, (1, P), 1)
    pxf = (w0 + pp % TILE).astype(jnp.float32)
    pyf = (h0 + pp // TILE).astype(jnp.float32)

    tril = (jax.lax.broadcasted_iota(jnp.int32, (K2, K2), 1) <
            jax.lax.broadcasted_iota(jnp.int32, (K2, K2), 0)).astype(jnp.float32)
    gidx = jax.lax.broadcasted_iota(jnp.int32, (K2, 1), 0)

    def chunk_body(c, carry):
        t_carry, acc_r, acc_g, acc_b, acc_a = carry
        blk = jnp.transpose(packed_ref[0, c])
        mx = blk[:, 0:1]
        my = blk[:, 1:2]
        i00 = blk[:, 2:3]
        i11 = blk[:, 3:4]
        i01 = blk[:, 4:5]
        op = blk[:, 5:6]
        colr = blk[:, 6:7]
        colg = blk[:, 7:8]
        colb = blk[:, 8:9]
        valid = (gidx + c * K2) < cnt
        dx = pxf - mx
        dy = pyf - my
        quad = dx * dx * i00 + dy * dy * i11 + 2.0 * (dx * dy) * i01
        gw = jnp.exp(-0.5 * quad)
        alpha = jnp.where(valid, jnp.minimum(gw * op, 0.99), 0.0)
        lg = jnp.log1p(-alpha)
        sexc = jax.lax.dot_general(
            tril, lg, (((1,), (0,)), ((), ())),
            precision=jax.lax.Precision.HIGHEST,
            preferred_element_type=jnp.float32)
        wgt = t_carry * jnp.exp(sexc) * alpha
        acc_r = acc_r + jnp.sum(wgt * colr, axis=0, keepdims=True)
        acc_g = acc_g + jnp.sum(wgt * colg, axis=0, keepdims=True)
        acc_b = acc_b + jnp.sum(wgt * colb, axis=0, keepdims=True)
        acc_a = acc_a + jnp.sum(wgt, axis=0, keepdims=True)
        t_carry = t_carry * jnp.exp(jnp.sum(lg, axis=0, keepdims=True))
        return t_carry, acc_r, acc_g, acc_b, acc_a

    init = (jnp.ones((1, P), jnp.float32),) + \
           tuple(jnp.zeros((1, P), jnp.float32) for _ in range(4))
    _, acc_r, acc_g, acc_b, acc_a = jax.lax.fori_loop(0, nch, chunk_body, init)

    resid = (1.0 - acc_a) * BKGD
    out_ref[0, 0:1, :] = acc_r + resid
    out_ref[0, 1:2, :] = acc_g + resid
    out_ref[0, 2:3, :] = acc_b + resid
    out_ref[0, 3:4, :] = acc_a


def _raster(cnt, packed):
    grid_spec = pltpu.PrefetchScalarGridSpec(
        num_scalar_prefetch=1,
        grid=(NT,),
        in_specs=[
            pl.BlockSpec((1, NCH, 16, K2), lambda t, cnt_ref: (t, 0, 0, 0)),
        ],
        out_specs=[
            pl.BlockSpec((1, 8, P), lambda t, cnt_ref: (t, 0, 0)),
        ],
    )
    return pl.pallas_call(
        _raster_kernel,
        grid_spec=grid_spec,
        out_shape=[jax.ShapeDtypeStruct((NT, 8, P), jnp.float32)],
    )(cnt, packed)[0]


@jax.jit
def kernel(means2D, cov2d, color, opacity, depths):
    order = jnp.argsort(depths)
    attrs = jnp.stack([
        means2D[:, 0], means2D[:, 1],
        cov2d[:, 0, 0], cov2d[:, 0, 1], cov2d[:, 1, 1],
        opacity[:, 0],
        color[:, 0], color[:, 1], color[:, 2],
    ], axis=0)[:, order]
    attrs = jnp.concatenate(
        [attrs, jnp.zeros((16 - attrs.shape[0], N), jnp.float32)], axis=0)
    covu = jnp.stack([cov2d[:, 0, 0], cov2d[:, 0, 1], cov2d[:, 1, 1]], axis=0)
    covu = jnp.concatenate(
        [covu, jnp.zeros((8 - covu.shape[0], N), jnp.float32)], axis=0)

    prep, rects, rad = _prep(attrs, covu)
    tt = jnp.arange(NT, dtype=jnp.int32)
    origins = jnp.stack([
        jnp.broadcast_to(((tt % NTX) * TILE)[:, None], (NT, L)),
        jnp.broadcast_to(((tt // NTX) * TILE)[:, None], (NT, L)),
    ], axis=1).astype(jnp.float32)
    packed, counts = _binning(rects, prep, origins)
    cnt = counts[:, 0]
    out = _raster(cnt, packed)
    img = out[:, :4, :].reshape(NTY, NTX, 4, TILE, TILE)
    img = jnp.transpose(img, (0, 3, 1, 4, 2)).reshape(H, W, 4)
    return img[:, :, :3], img[:, :, 3:4], rad[0]


# speedup vs baseline: 3.2047x; 1.1011x over previous
import functools

import jax
import jax.numpy as jnp
from jax.experimental import pallas as pl
from jax.experimental.pallas import tpu as pltpu
from jax.experimental.pallas import tpu_sc as plsc

H = 128
W = 128
TILE = 16
N = 1024
NTX = W // TILE
NTY = H // TILE
NT = NTX * NTY
K2 = 128
NCH = N // K2
P = TILE * TILE
BKGD = 1.0


def _prep_kernel(attrs_ref, covu_ref, prep_ref, rects_ref, radii_ref):
    ca = covu_ref[0:1, :]
    cb = covu_ref[1:2, :]
    cd = covu_ref[2:3, :]
    det = ca * cd - cb * cb
    mid = 0.5 * (ca + cd)
    root = jnp.sqrt(jnp.maximum(mid * mid - det, 0.1))
    radii_ref[0:1, :] = jnp.ceil(3.0 * jnp.sqrt(mid + root))

    mx = attrs_ref[0:1, :]
    my = attrs_ref[1:2, :]
    ca = attrs_ref[2:3, :]
    cb = attrs_ref[3:4, :]
    cd = attrs_ref[4:5, :]
    det = ca * cd - cb * cb
    prep_ref[0:1, :] = mx
    prep_ref[1:2, :] = my
    prep_ref[2:3, :] = cd / det
    prep_ref[3:4, :] = ca / det
    prep_ref[4:5, :] = -cb / det
    prep_ref[5:16, :] = attrs_ref[5:16, :]

    mid = 0.5 * (ca + cd)
    root = jnp.sqrt(jnp.maximum(mid * mid - det, 0.1))
    rad = jnp.ceil(3.0 * jnp.sqrt(mid + root))
    rects_ref[0:1, :] = jnp.clip(mx - rad, 0.0, W - 1.0)
    rects_ref[1:2, :] = jnp.clip(mx + rad, 0.0, W - 1.0)
    rects_ref[2:3, :] = jnp.clip(my - rad, 0.0, H - 1.0)
    rects_ref[3:4, :] = jnp.clip(my + rad, 0.0, H - 1.0)
    rects_ref[4:8, :] = jnp.zeros((4, N), jnp.float32)


def _prep(attrs, covu):
    return pl.pallas_call(
        _prep_kernel,
        out_shape=[
            jax.ShapeDtypeStruct((16, N), jnp.float32),
            jax.ShapeDtypeStruct((8, N), jnp.float32),
            jax.ShapeDtypeStruct((1, N), jnp.float32),
        ],
    )(attrs, covu)


NC = 2
NS = 16
NWORK = NC * NS
L = 16


def _bin_kernel(rects_hbm, prep_hbm, orig_hbm, packed_hbm, counts_hbm,
                rects_v, prep_v, idx_v, pk_v, cnt_v, orig_v, base_v, vals_v):
    wid = jax.lax.axis_index("s") * NC + jax.lax.axis_index("c")
    pltpu.sync_copy(rects_hbm, rects_v)
    pltpu.sync_copy(prep_hbm, prep_v)
    lanes = jax.lax.iota(jnp.int32, L)

    for j in range(NT // NWORK):
        t = wid * (NT // NWORK) + j
        pltpu.sync_copy(orig_hbm.at[t], orig_v)

        def zero_body(i, _):
            idx_v[pl.ds(i * L, L)] = jnp.zeros((L,), jnp.int32)
            return 0
        jax.lax.fori_loop(0, N // L, zero_body, 0)

        base_v[...] = jnp.zeros((L,), jnp.int32)
        vals_v[...] = lanes

        def bin_body(g, _):
            base = base_v[...]
            vals = vals_v[...]
            wxv = orig_v[0, :]
            hyv = orig_v[1, :]
            rminx = rects_v[0, pl.ds(g * L, L)]
            rmaxx = rects_v[1, pl.ds(g * L, L)]
            rminy = rects_v[2, pl.ds(g * L, L)]
            rmaxy = rects_v[3, pl.ds(g * L, L)]
            m1 = (jnp.minimum(rmaxx, wxv + (TILE - 1.0)) >
                  jnp.maximum(rminx, wxv))
            m2 = (jnp.minimum(rmaxy, hyv + (TILE - 1.0)) >
                  jnp.maximum(rminy, hyv))
            ones = jnp.full((L,), 1, jnp.int32)
            zeros = jnp.full((L,), 0, jnp.int32)
            mi = jnp.where(m1, ones, zeros) * jnp.where(m2, ones, zeros)
            m = mi > zeros
            pos = base + plsc.cumsum(mi) - mi
            plsc.store_scatter(idx_v, [pos], vals, mask=m)
            base_v[...] = base + plsc.all_reduce_population_count(m)
            vals_v[...] = vals + L
            return 0
        jax.lax.fori_loop(0, N // L, bin_body, 0)
        base = base_v[...]

        vals_v[...] = lanes

        def gat_body(g2, _):
            gidx = vals_v[...]
            members = idx_v[pl.ds(g2 * L, L)]
            for a in range(9):
                asplat = jnp.full((L,), a, jnp.int32)
                vals = plsc.load_gather(prep_v, [asplat, members])
                plsc.store_scatter(pk_v, [asplat, gidx], vals)
            vals_v[...] = gidx + L
            return 0
        jax.lax.fori_loop(0, N // L, gat_body, 0)

        for c in range(NCH):
            pltpu.sync_copy(pk_v.at[:, pl.ds(c * K2, K2)], packed_hbm.at[t, c])
        cnt_v[...] = base
        pltpu.sync_copy(cnt_v, counts_hbm.at[t])


def _binning(rects, prep, origins):
    mesh = plsc.VectorSubcoreMesh(core_axis_name="c", subcore_axis_name="s")
    run = pl.kernel(
        _bin_kernel, mesh=mesh,
        compiler_params=pltpu.CompilerParams(needs_layout_passes=False),
        out_type=[
            jax.ShapeDtypeStruct((NT, NCH, 16, K2), jnp.float32),
            jax.ShapeDtypeStruct((NT, L), jnp.int32),
        ],
        scratch_types=[
            pltpu.VMEM((8, N), jnp.float32),
            pltpu.VMEM((16, N), jnp.float32),
            pltpu.VMEM((N,), jnp.int32),
            pltpu.VMEM((16, N), jnp.float32),
            pltpu.VMEM((L,), jnp.int32),
            pltpu.VMEM((2, L), jnp.float32),
            pltpu.VMEM((L,), jnp.int32),
            pltpu.VMEM((L,), jnp.int32),
        ],
    )
    return run(rects, prep, origins)


def _raster_kernel(cnt_ref, packed_ref, out_ref):
    t = pl.program_id(0)
    cnt = cnt_ref[t]
    nch = (cnt + K2 - 1) // K2
    w0 = (t % NTX) * TILE
    h0 = (t // NTX) * TILE

    pp = jax.lax.broadcasted_iota(jnp.int32, (1, P), 1)
    pxf = (w0 + pp % TILE).astype(jnp.float32)
    pyf = (h0 + pp // TILE).astype(jnp.float32)

    tril = (jax.lax.broadcasted_iota(jnp.int32, (K2, K2), 1) <
            jax.lax.broadcasted_iota(jnp.int32, (K2, K2), 0)).astype(jnp.float32)
    gidx = jax.lax.broadcasted_iota(jnp.int32, (K2, 1), 0)

    def chunk_body(c, carry):
        t_carry, acc_r, acc_g, acc_b, acc_a = carry
        blk = jnp.transpose(packed_ref[0, c])
        mx = blk[:, 0:1]
        my = blk[:, 1:2]
        i00 = blk[:, 2:3]
        i11 = blk[:, 3:4]
        i01 = blk[:, 4:5]
        op = blk[:, 5:6]
        colr = blk[:, 6:7]
        colg = blk[:, 7:8]
        colb = blk[:, 8:9]
        valid = (gidx + c * K2) < cnt
        dx = pxf - mx
        dy = pyf - my
        quad = dx * dx * i00 + dy * dy * i11 + 2.0 * (dx * dy) * i01
        gw = jnp.exp(-0.5 * quad)
        alpha = jnp.where(valid, jnp.minimum(gw * op, 0.99), 0.0)
        lg = jnp.log1p(-alpha)
        sexc = jax.lax.dot_general(
            tril, lg, (((1,), (0,)), ((), ())),
            precision=jax.lax.Precision.HIGHEST,
            preferred_element_type=jnp.float32)
        wgt = t_carry * jnp.exp(sexc) * alpha
        acc_r = acc_r + jnp.sum(wgt * colr, axis=0, keepdims=True)
        acc_g = acc_g + jnp.sum(wgt * colg, axis=0, keepdims=True)
        acc_b = acc_b + jnp.sum(wgt * colb, axis=0, keepdims=True)
        acc_a = acc_a + jnp.sum(wgt, axis=0, keepdims=True)
        t_carry = t_carry * jnp.exp(jnp.sum(lg, axis=0, keepdims=True))
        return t_carry, acc_r, acc_g, acc_b, acc_a

    init = (jnp.ones((1, P), jnp.float32),) + \
           tuple(jnp.zeros((1, P), jnp.float32) for _ in range(4))
    _, acc_r, acc_g, acc_b, acc_a = jax.lax.fori_loop(0, nch, chunk_body, init)

    resid = (1.0 - acc_a) * BKGD
    out_ref[0, 0:1, :] = acc_r + resid
    out_ref[0, 1:2, :] = acc_g + resid
    out_ref[0, 2:3, :] = acc_b + resid
    out_ref[0, 3:4, :] = acc_a


def _raster(cnt, packed):
    grid_spec = pltpu.PrefetchScalarGridSpec(
        num_scalar_prefetch=1,
        grid=(NT,),
        in_specs=[
            pl.BlockSpec((1, NCH, 16, K2), lambda t, cnt_ref: (t, 0, 0, 0)),
        ],
        out_specs=[
            pl.BlockSpec((1, 8, P), lambda t, cnt_ref: (t, 0, 0)),
        ],
    )
    return pl.pallas_call(
        _raster_kernel,
        grid_spec=grid_spec,
        out_shape=[jax.ShapeDtypeStruct((NT, 8, P), jnp.float32)],
    )(cnt, packed)[0]


@jax.jit
def kernel(means2D, cov2d, color, opacity, depths):
    order = jnp.argsort(depths)
    attrs = jnp.stack([
        means2D[:, 0], means2D[:, 1],
        cov2d[:, 0, 0], cov2d[:, 0, 1], cov2d[:, 1, 1],
        opacity[:, 0],
        color[:, 0], color[:, 1], color[:, 2],
    ], axis=0)[:, order]
    attrs = jnp.concatenate(
        [attrs, jnp.zeros((16 - attrs.shape[0], N), jnp.float32)], axis=0)
    covu = jnp.stack([cov2d[:, 0, 0], cov2d[:, 0, 1], cov2d[:, 1, 1]], axis=0)
    covu = jnp.concatenate(
        [covu, jnp.zeros((8 - covu.shape[0], N), jnp.float32)], axis=0)

    prep, rects, rad = _prep(attrs, covu)
    tt = jnp.arange(NT, dtype=jnp.int32)
    origins = jnp.stack([
        jnp.broadcast_to(((tt % NTX) * TILE)[:, None], (NT, L)),
        jnp.broadcast_to(((tt // NTX) * TILE)[:, None], (NT, L)),
    ], axis=1).astype(jnp.float32)
    packed, counts = _binning(rects, prep, origins)
    cnt = counts[:, 0]
    out = _raster(cnt, packed)
    img = out[:, :4, :].reshape(NTY, NTX, 4, TILE, TILE)
    img = jnp.transpose(img, (0, 3, 1, 4, 2)).reshape(H, W, 4)
    return img[:, :, :3], img[:, :, 3:4], rad[0]


# speedup vs baseline: 3.4550x; 1.0781x over previous
import functools

import jax
import jax.numpy as jnp
from jax.experimental import pallas as pl
from jax.experimental.pallas import tpu as pltpu
from jax.experimental.pallas import tpu_sc as plsc

H = 128
W = 128
TILE = 16
N = 1024
NTX = W // TILE
NTY = H // TILE
NT = NTX * NTY
K2 = 128
NCH = N // K2
P = TILE * TILE
BKGD = 1.0


def _prep_kernel(attrs_ref, covu_ref, prep_ref, rects_ref, radii_ref):
    ca = covu_ref[0:1, :]
    cb = covu_ref[1:2, :]
    cd = covu_ref[2:3, :]
    det = ca * cd - cb * cb
    mid = 0.5 * (ca + cd)
    root = jnp.sqrt(jnp.maximum(mid * mid - det, 0.1))
    radii_ref[0:1, :] = jnp.ceil(3.0 * jnp.sqrt(mid + root))

    mx = attrs_ref[0:1, :]
    my = attrs_ref[1:2, :]
    ca = attrs_ref[2:3, :]
    cb = attrs_ref[3:4, :]
    cd = attrs_ref[4:5, :]
    det = ca * cd - cb * cb
    prep_ref[0:1, :] = mx
    prep_ref[1:2, :] = my
    prep_ref[2:3, :] = cd / det
    prep_ref[3:4, :] = ca / det
    prep_ref[4:5, :] = -cb / det
    prep_ref[5:16, :] = attrs_ref[5:16, :]

    mid = 0.5 * (ca + cd)
    root = jnp.sqrt(jnp.maximum(mid * mid - det, 0.1))
    rad = jnp.ceil(3.0 * jnp.sqrt(mid + root))
    rects_ref[0:1, :] = jnp.clip(mx - rad, 0.0, W - 1.0)
    rects_ref[1:2, :] = jnp.clip(mx + rad, 0.0, W - 1.0)
    rects_ref[2:3, :] = jnp.clip(my - rad, 0.0, H - 1.0)
    rects_ref[3:4, :] = jnp.clip(my + rad, 0.0, H - 1.0)
    rects_ref[4:8, :] = jnp.zeros((4, N), jnp.float32)


def _prep(attrs, covu):
    return pl.pallas_call(
        _prep_kernel,
        out_shape=[
            jax.ShapeDtypeStruct((16, N), jnp.float32),
            jax.ShapeDtypeStruct((8, N), jnp.float32),
            jax.ShapeDtypeStruct((1, N), jnp.float32),
        ],
    )(attrs, covu)


NC = 2
NS = 16
NWORK = NC * NS
L = 16


def _bin_kernel(rects_hbm, prep_hbm, orig_hbm, packed_hbm, counts_hbm,
                rects_v, prep_v, idx_v, pk_v, cnt_v, orig_v, base_v, vals_v):
    wid = jax.lax.axis_index("s") * NC + jax.lax.axis_index("c")
    pltpu.sync_copy(rects_hbm, rects_v)
    pltpu.sync_copy(prep_hbm, prep_v)
    lanes = jax.lax.iota(jnp.int32, L)

    for j in range(NT // NWORK):
        t = wid * (NT // NWORK) + j
        pltpu.sync_copy(orig_hbm.at[t], orig_v)
        base_v[...] = jnp.zeros((L,), jnp.int32)
        vals_v[...] = lanes

        def bin_body(g, _):
            base = base_v[...]
            vals = vals_v[...]
            wxv = orig_v[0, :]
            hyv = orig_v[1, :]
            rminx = rects_v[0, pl.ds(g * L, L)]
            rmaxx = rects_v[1, pl.ds(g * L, L)]
            rminy = rects_v[2, pl.ds(g * L, L)]
            rmaxy = rects_v[3, pl.ds(g * L, L)]
            m1 = (jnp.minimum(rmaxx, wxv + (TILE - 1.0)) >
                  jnp.maximum(rminx, wxv))
            m2 = (jnp.minimum(rmaxy, hyv + (TILE - 1.0)) >
                  jnp.maximum(rminy, hyv))
            ones = jnp.full((L,), 1, jnp.int32)
            zeros = jnp.full((L,), 0, jnp.int32)
            mi = jnp.where(m1, ones, zeros) * jnp.where(m2, ones, zeros)
            m = mi > zeros
            pos = base + plsc.cumsum(mi) - mi
            plsc.store_scatter(idx_v, [pos], vals, mask=m)
            base_v[...] = base + plsc.all_reduce_population_count(m)
            vals_v[...] = vals + L
            return 0
        jax.lax.fori_loop(0, N // L, bin_body, 0)
        base = base_v[...]
        cnt = jnp.max(base)

        vals_v[...] = lanes

        def gat_body(g2, _):
            gidx = vals_v[...]
            members = idx_v[pl.ds(g2 * L, L)]
            members = jnp.minimum(jnp.maximum(members, jnp.full((L,), 0, jnp.int32)),
                                  jnp.full((L,), N - 1, jnp.int32))
            for a in range(9):
                asplat = jnp.full((L,), a, jnp.int32)
                vals = plsc.load_gather(prep_v, [asplat, members])
                plsc.store_scatter(pk_v, [asplat, gidx], vals)
            vals_v[...] = gidx + L
            return 0
        jax.lax.fori_loop(0, (cnt + L - 1) // L, gat_body, 0)

        nch = (cnt + K2 - 1) // K2
        for c in range(NCH):
            @pl.when(c < nch)
            def _():
                pltpu.sync_copy(pk_v.at[:, pl.ds(c * K2, K2)],
                                packed_hbm.at[t, c])
        cnt_v[...] = base
        pltpu.sync_copy(cnt_v, counts_hbm.at[t])


def _binning(rects, prep, origins):
    mesh = plsc.VectorSubcoreMesh(core_axis_name="c", subcore_axis_name="s")
    run = pl.kernel(
        _bin_kernel, mesh=mesh,
        compiler_params=pltpu.CompilerParams(needs_layout_passes=False),
        out_type=[
            jax.ShapeDtypeStruct((NT, NCH, 16, K2), jnp.float32),
            jax.ShapeDtypeStruct((NT, L), jnp.int32),
        ],
        scratch_types=[
            pltpu.VMEM((8, N), jnp.float32),
            pltpu.VMEM((16, N), jnp.float32),
            pltpu.VMEM((N,), jnp.int32),
            pltpu.VMEM((16, N), jnp.float32),
            pltpu.VMEM((L,), jnp.int32),
            pltpu.VMEM((2, L), jnp.float32),
            pltpu.VMEM((L,), jnp.int32),
            pltpu.VMEM((L,), jnp.int32),
        ],
    )
    return run(rects, prep, origins)


def _raster_kernel(cnt_ref, packed_ref, out_ref):
    t = pl.program_id(0)
    cnt = cnt_ref[t]
    nch = (cnt + K2 - 1) // K2
    w0 = (t % NTX) * TILE
    h0 = (t // NTX) * TILE

    pp = jax.lax.broadcasted_iota(jnp.int32, (1, P), 1)
    pxf = (w0 + pp % TILE).astype(jnp.float32)
    pyf = (h0 + pp // TILE).astype(jnp.float32)

    tril = (jax.lax.broadcasted_iota(jnp.int32, (K2, K2), 1) <
            jax.lax.broadcasted_iota(jnp.int32, (K2, K2), 0)).astype(jnp.float32)
    gidx = jax.lax.broadcasted_iota(jnp.int32, (K2, 1), 0)

    def chunk_body(c, carry):
        t_carry, acc_r, acc_g, acc_b, acc_a = carry
        blk = jnp.transpose(packed_ref[0, c])
        mx = blk[:, 0:1]
        my = blk[:, 1:2]
        i00 = blk[:, 2:3]
        i11 = blk[:, 3:4]
        i01 = blk[:, 4:5]
        op = blk[:, 5:6]
        colr = blk[:, 6:7]
        colg = blk[:, 7:8]
        colb = blk[:, 8:9]
        valid = (gidx + c * K2) < cnt
        dx = pxf - mx
        dy = pyf - my
        quad = dx * dx * i00 + dy * dy * i11 + 2.0 * (dx * dy) * i01
        gw = jnp.exp(-0.5 * quad)
        alpha = jnp.where(valid, jnp.minimum(gw * op, 0.99), 0.0)
        lg = jnp.log1p(-alpha)
        sexc = jax.lax.dot_general(
            tril, lg, (((1,), (0,)), ((), ())),
            precision=jax.lax.Precision.HIGHEST,
            preferred_element_type=jnp.float32)
        wgt = t_carry * jnp.exp(sexc) * alpha
        acc_r = acc_r + jnp.sum(wgt * colr, axis=0, keepdims=True)
        acc_g = acc_g + jnp.sum(wgt * colg, axis=0, keepdims=True)
        acc_b = acc_b + jnp.sum(wgt * colb, axis=0, keepdims=True)
        acc_a = acc_a + jnp.sum(wgt, axis=0, keepdims=True)
        t_carry = t_carry * jnp.exp(jnp.sum(lg, axis=0, keepdims=True))
        return t_carry, acc_r, acc_g, acc_b, acc_a

    init = (jnp.ones((1, P), jnp.float32),) + \
           tuple(jnp.zeros((1, P), jnp.float32) for _ in range(4))
    _, acc_r, acc_g, acc_b, acc_a = jax.lax.fori_loop(0, nch, chunk_body, init)

    resid = (1.0 - acc_a) * BKGD
    out_ref[0, 0:1, :] = acc_r + resid
    out_ref[0, 1:2, :] = acc_g + resid
    out_ref[0, 2:3, :] = acc_b + resid
    out_ref[0, 3:4, :] = acc_a


def _raster(cnt, packed):
    grid_spec = pltpu.PrefetchScalarGridSpec(
        num_scalar_prefetch=1,
        grid=(NT,),
        in_specs=[
            pl.BlockSpec((1, NCH, 16, K2), lambda t, cnt_ref: (t, 0, 0, 0)),
        ],
        out_specs=[
            pl.BlockSpec((1, 8, P), lambda t, cnt_ref: (t, 0, 0)),
        ],
    )
    return pl.pallas_call(
        _raster_kernel,
        grid_spec=grid_spec,
        out_shape=[jax.ShapeDtypeStruct((NT, 8, P), jnp.float32)],
    )(cnt, packed)[0]


@jax.jit
def kernel(means2D, cov2d, color, opacity, depths):
    order = jnp.argsort(depths)
    attrs = jnp.stack([
        means2D[:, 0], means2D[:, 1],
        cov2d[:, 0, 0], cov2d[:, 0, 1], cov2d[:, 1, 1],
        opacity[:, 0],
        color[:, 0], color[:, 1], color[:, 2],
    ], axis=0)[:, order]
    attrs = jnp.concatenate(
        [attrs, jnp.zeros((16 - attrs.shape[0], N), jnp.float32)], axis=0)
    covu = jnp.stack([cov2d[:, 0, 0], cov2d[:, 0, 1], cov2d[:, 1, 1]], axis=0)
    covu = jnp.concatenate(
        [covu, jnp.zeros((8 - covu.shape[0], N), jnp.float32)], axis=0)

    prep, rects, rad = _prep(attrs, covu)
    tt = jnp.arange(NT, dtype=jnp.int32)
    origins = jnp.stack([
        jnp.broadcast_to(((tt % NTX) * TILE)[:, None], (NT, L)),
        jnp.broadcast_to(((tt // NTX) * TILE)[:, None], (NT, L)),
    ], axis=1).astype(jnp.float32)
    packed, counts = _binning(rects, prep, origins)
    cnt = counts[:, 0]
    out = _raster(cnt, packed)
    img = out[:, :4, :].reshape(NTY, NTX, 4, TILE, TILE)
    img = jnp.transpose(img, (0, 3, 1, 4, 2)).reshape(H, W, 4)
    return img[:, :, :3], img[:, :, 3:4], rad[0]


# speedup vs baseline: 3.5902x; 1.0391x over previous
import functools

import jax
import jax.numpy as jnp
from jax.experimental import pallas as pl
from jax.experimental.pallas import tpu as pltpu
from jax.experimental.pallas import tpu_sc as plsc

H = 128
W = 128
TILE = 16
N = 1024
NTX = W // TILE
NTY = H // TILE
NT = NTX * NTY
K2 = 128
NCH = N // K2
P = TILE * TILE
BKGD = 1.0


def _prep_kernel(attrs_ref, covu_ref, prep_ref, rects_ref, radii_ref):
    ca = covu_ref[0:1, :]
    cb = covu_ref[1:2, :]
    cd = covu_ref[2:3, :]
    det = ca * cd - cb * cb
    mid = 0.5 * (ca + cd)
    root = jnp.sqrt(jnp.maximum(mid * mid - det, 0.1))
    radii_ref[0:1, :] = jnp.ceil(3.0 * jnp.sqrt(mid + root))

    mx = attrs_ref[0:1, :]
    my = attrs_ref[1:2, :]
    ca = attrs_ref[2:3, :]
    cb = attrs_ref[3:4, :]
    cd = attrs_ref[4:5, :]
    det = ca * cd - cb * cb
    prep_ref[0:1, :] = mx
    prep_ref[1:2, :] = my
    prep_ref[2:3, :] = cd / det
    prep_ref[3:4, :] = ca / det
    prep_ref[4:5, :] = -cb / det
    prep_ref[5:16, :] = attrs_ref[5:16, :]

    mid = 0.5 * (ca + cd)
    root = jnp.sqrt(jnp.maximum(mid * mid - det, 0.1))
    rad = jnp.ceil(3.0 * jnp.sqrt(mid + root))
    rects_ref[0:1, :] = jnp.clip(mx - rad, 0.0, W - 1.0)
    rects_ref[1:2, :] = jnp.clip(mx + rad, 0.0, W - 1.0)
    rects_ref[2:3, :] = jnp.clip(my - rad, 0.0, H - 1.0)
    rects_ref[3:4, :] = jnp.clip(my + rad, 0.0, H - 1.0)
    rects_ref[4:8, :] = jnp.zeros((4, N), jnp.float32)


def _prep(attrs, covu):
    return pl.pallas_call(
        _prep_kernel,
        out_shape=[
            jax.ShapeDtypeStruct((16, N), jnp.float32),
            jax.ShapeDtypeStruct((8, N), jnp.float32),
            jax.ShapeDtypeStruct((1, N), jnp.float32),
        ],
    )(attrs, covu)


NC = 2
NS = 16
NWORK = NC * NS
L = 16


def _bin_kernel(rects_hbm, prep_hbm, orig_hbm, packed_hbm, counts_hbm,
                rects_v, prep_v, idx_v, pk_v, cnt_v, orig_v, base_v, vals_v):
    wid = jax.lax.axis_index("s") * NC + jax.lax.axis_index("c")
    pltpu.sync_copy(rects_hbm, rects_v)
    pltpu.sync_copy(prep_hbm, prep_v)
    lanes = jax.lax.iota(jnp.int32, L)

    for j in range(NT // NWORK):
        t = wid * (NT // NWORK) + j
        pltpu.sync_copy(orig_hbm.at[t], orig_v)
        base_v[...] = jnp.zeros((L,), jnp.int32)
        vals_v[...] = lanes

        def bin_body(g, _):
            base = base_v[...]
            vals = vals_v[...]
            wxv = orig_v[0, :]
            hyv = orig_v[1, :]
            rminx = rects_v[0, pl.ds(g * L, L)]
            rmaxx = rects_v[1, pl.ds(g * L, L)]
            rminy = rects_v[2, pl.ds(g * L, L)]
            rmaxy = rects_v[3, pl.ds(g * L, L)]
            m1 = (jnp.minimum(rmaxx, wxv + (TILE - 1.0)) >
                  jnp.maximum(rminx, wxv))
            m2 = (jnp.minimum(rmaxy, hyv + (TILE - 1.0)) >
                  jnp.maximum(rminy, hyv))
            ones = jnp.full((L,), 1, jnp.int32)
            zeros = jnp.full((L,), 0, jnp.int32)
            mi = jnp.where(m1, ones, zeros) * jnp.where(m2, ones, zeros)
            m = mi > zeros
            pos = base + plsc.cumsum(mi) - mi
            plsc.store_scatter(idx_v, [pos], vals, mask=m)
            base_v[...] = base + plsc.all_reduce_population_count(m)
            vals_v[...] = vals + L
            return 0
        jax.lax.fori_loop(0, N // L, bin_body, 0)
        base = base_v[...]
        cnt = jnp.max(base)

        vals_v[...] = lanes

        def gat_body(g2, _):
            gidx = vals_v[...]
            members = idx_v[pl.ds(g2 * L, L)]
            members = jnp.minimum(jnp.maximum(members, jnp.full((L,), 0, jnp.int32)),
                                  jnp.full((L,), N - 1, jnp.int32))
            for a in range(9):
                asplat = jnp.full((L,), a, jnp.int32)
                vals = plsc.load_gather(prep_v, [asplat, members])
                plsc.store_scatter(pk_v, [asplat, gidx], vals)
            vals_v[...] = gidx + L
            return 0
        jax.lax.fori_loop(0, (cnt + L - 1) // L, gat_body, 0)

        nch = (cnt + K2 - 1) // K2
        for c in range(NCH):
            @pl.when(c < nch)
            def _():
                pltpu.sync_copy(pk_v.at[:, pl.ds(c * K2, K2)],
                                packed_hbm.at[t, c])
        cnt_v[...] = base
        pltpu.sync_copy(cnt_v, counts_hbm.at[t])


def _binning(rects, prep, origins):
    mesh = plsc.VectorSubcoreMesh(core_axis_name="c", subcore_axis_name="s")
    run = pl.kernel(
        _bin_kernel, mesh=mesh,
        compiler_params=pltpu.CompilerParams(needs_layout_passes=False),
        out_type=[
            jax.ShapeDtypeStruct((NT, NCH, 16, K2), jnp.float32),
            jax.ShapeDtypeStruct((NT, L), jnp.int32),
        ],
        scratch_types=[
            pltpu.VMEM((8, N), jnp.float32),
            pltpu.VMEM((16, N), jnp.float32),
            pltpu.VMEM((N,), jnp.int32),
            pltpu.VMEM((16, N), jnp.float32),
            pltpu.VMEM((L,), jnp.int32),
            pltpu.VMEM((2, L), jnp.float32),
            pltpu.VMEM((L,), jnp.int32),
            pltpu.VMEM((L,), jnp.int32),
        ],
    )
    return run(rects, prep, origins)


def _raster_kernel(cnt_ref, packed_ref, out_ref):
    t = pl.program_id(0)
    cnt = cnt_ref[t]
    nch = (cnt + K2 - 1) // K2
    w0 = (t % NTX) * TILE
    h0 = (t // NTX) * TILE

    pp = jax.lax.broadcasted_iota(jnp.int32, (1, P), 1)
    pxf = (w0 + pp % TILE).astype(jnp.float32)
    pyf = (h0 + pp // TILE).astype(jnp.float32)

    tril = (jax.lax.broadcasted_iota(jnp.int32, (K2, K2), 1) <
            jax.lax.broadcasted_iota(jnp.int32, (K2, K2), 0)).astype(jnp.float32)
    gidx = jax.lax.broadcasted_iota(jnp.int32, (K2, 1), 0)

    def chunk_body(c, carry):
        t_carry, acc_r, acc_g, acc_b, acc_a = carry
        blk = jnp.transpose(packed_ref[0, c])
        mx = blk[:, 0:1]
        my = blk[:, 1:2]
        i00 = blk[:, 2:3]
        i11 = blk[:, 3:4]
        i01 = blk[:, 4:5]
        op = blk[:, 5:6]
        colr = blk[:, 6:7]
        colg = blk[:, 7:8]
        colb = blk[:, 8:9]
        valid = (gidx + c * K2) < cnt
        dx = pxf - mx
        dy = pyf - my
        quad = dx * dx * i00 + dy * dy * i11 + 2.0 * (dx * dy) * i01
        gw = jnp.exp(-0.5 * quad)
        alpha = jnp.where(valid, jnp.minimum(gw * op, 0.99), 0.0)
        lg = jnp.log1p(-alpha)
        sexc = jax.lax.dot_general(
            tril, lg, (((1,), (0,)), ((), ())),
            precision=jax.lax.Precision.DEFAULT,
            preferred_element_type=jnp.float32)
        wgt = t_carry * jnp.exp(sexc) * alpha
        acc_r = acc_r + jnp.sum(wgt * colr, axis=0, keepdims=True)
        acc_g = acc_g + jnp.sum(wgt * colg, axis=0, keepdims=True)
        acc_b = acc_b + jnp.sum(wgt * colb, axis=0, keepdims=True)
        acc_a = acc_a + jnp.sum(wgt, axis=0, keepdims=True)
        t_carry = t_carry * jnp.exp(jnp.sum(lg, axis=0, keepdims=True))
        return t_carry, acc_r, acc_g, acc_b, acc_a

    init = (jnp.ones((1, P), jnp.float32),) + \
           tuple(jnp.zeros((1, P), jnp.float32) for _ in range(4))
    _, acc_r, acc_g, acc_b, acc_a = jax.lax.fori_loop(0, nch, chunk_body, init)

    resid = (1.0 - acc_a) * BKGD
    out_ref[0, 0:1, :] = acc_r + resid
    out_ref[0, 1:2, :] = acc_g + resid
    out_ref[0, 2:3, :] = acc_b + resid
    out_ref[0, 3:4, :] = acc_a


def _raster(cnt, packed):
    grid_spec = pltpu.PrefetchScalarGridSpec(
        num_scalar_prefetch=1,
        grid=(NT,),
        in_specs=[
            pl.BlockSpec((1, NCH, 16, K2), lambda t, cnt_ref: (t, 0, 0, 0)),
        ],
        out_specs=[
            pl.BlockSpec((1, 8, P), lambda t, cnt_ref: (t, 0, 0)),
        ],
    )
    return pl.pallas_call(
        _raster_kernel,
        grid_spec=grid_spec,
        out_shape=[jax.ShapeDtypeStruct((NT, 8, P), jnp.float32)],
    )(cnt, packed)[0]


@jax.jit
def kernel(means2D, cov2d, color, opacity, depths):
    order = jnp.argsort(depths)
    attrs = jnp.stack([
        means2D[:, 0], means2D[:, 1],
        cov2d[:, 0, 0], cov2d[:, 0, 1], cov2d[:, 1, 1],
        opacity[:, 0],
        color[:, 0], color[:, 1], color[:, 2],
    ], axis=0)[:, order]
    attrs = jnp.concatenate(
        [attrs, jnp.zeros((16 - attrs.shape[0], N), jnp.float32)], axis=0)
    covu = jnp.stack([cov2d[:, 0, 0], cov2d[:, 0, 1], cov2d[:, 1, 1]], axis=0)
    covu = jnp.concatenate(
        [covu, jnp.zeros((8 - covu.shape[0], N), jnp.float32)], axis=0)

    prep, rects, rad = _prep(attrs, covu)
    tt = jnp.arange(NT, dtype=jnp.int32)
    origins = jnp.stack([
        jnp.broadcast_to(((tt % NTX) * TILE)[:, None], (NT, L)),
        jnp.broadcast_to(((tt // NTX) * TILE)[:, None], (NT, L)),
    ], axis=1).astype(jnp.float32)
    packed, counts = _binning(rects, prep, origins)
    cnt = counts[:, 0]
    out = _raster(cnt, packed)
    img = out[:, :4, :].reshape(NTY, NTX, 4, TILE, TILE)
    img = jnp.transpose(img, (0, 3, 1, 4, 2)).reshape(H, W, 4)
    return img[:, :, :3], img[:, :, 3:4], rad[0]


# speedup vs baseline: 4.3685x; 1.2168x over previous
import functools

import jax
import jax.numpy as jnp
from jax.experimental import pallas as pl
from jax.experimental.pallas import tpu as pltpu
from jax.experimental.pallas import tpu_sc as plsc

H = 128
W = 128
TILE = 16
N = 1024
NTX = W // TILE
NTY = H // TILE
NT = NTX * NTY
K2 = 128
NCH = N // K2
P = TILE * TILE
BKGD = 1.0


def _prep_kernel(attrs_ref, covu_ref, prep_ref, rects_ref, radii_ref):
    ca = covu_ref[0:1, :]
    cb = covu_ref[1:2, :]
    cd = covu_ref[2:3, :]
    det = ca * cd - cb * cb
    mid = 0.5 * (ca + cd)
    root = jnp.sqrt(jnp.maximum(mid * mid - det, 0.1))
    radii_ref[0:1, :] = jnp.ceil(3.0 * jnp.sqrt(mid + root))

    mx = attrs_ref[0:1, :]
    my = attrs_ref[1:2, :]
    ca = attrs_ref[2:3, :]
    cb = attrs_ref[3:4, :]
    cd = attrs_ref[4:5, :]
    det = ca * cd - cb * cb
    prep_ref[0:1, :] = mx
    prep_ref[1:2, :] = my
    prep_ref[2:3, :] = cd / det
    prep_ref[3:4, :] = ca / det
    prep_ref[4:5, :] = -cb / det
    prep_ref[5:16, :] = attrs_ref[5:16, :]

    mid = 0.5 * (ca + cd)
    root = jnp.sqrt(jnp.maximum(mid * mid - det, 0.1))
    rad = jnp.ceil(3.0 * jnp.sqrt(mid + root))
    rects_ref[0:1, :] = jnp.clip(mx - rad, 0.0, W - 1.0)
    rects_ref[1:2, :] = jnp.clip(mx + rad, 0.0, W - 1.0)
    rects_ref[2:3, :] = jnp.clip(my - rad, 0.0, H - 1.0)
    rects_ref[3:4, :] = jnp.clip(my + rad, 0.0, H - 1.0)
    rects_ref[4:8, :] = jnp.zeros((4, N), jnp.float32)


def _prep(attrs, covu):
    return pl.pallas_call(
        _prep_kernel,
        out_shape=[
            jax.ShapeDtypeStruct((16, N), jnp.float32),
            jax.ShapeDtypeStruct((8, N), jnp.float32),
            jax.ShapeDtypeStruct((1, N), jnp.float32),
        ],
    )(attrs, covu)


NC = 2
NS = 16
NWORK = NC * NS
L = 16


def _bin_kernel(rects_hbm, prep_hbm, orig_hbm, packed_hbm, counts_hbm,
                rects_v, prep_v, idx_v, pk_v, cnt_v, orig_v, base_v, vals_v):
    wid = jax.lax.axis_index("s") * NC + jax.lax.axis_index("c")
    pltpu.sync_copy(rects_hbm, rects_v)
    pltpu.sync_copy(prep_hbm, prep_v)
    lanes = jax.lax.iota(jnp.int32, L)

    for j in range(NT // NWORK):
        t = wid * (NT // NWORK) + j
        pltpu.sync_copy(orig_hbm.at[t], orig_v)
        base_v[...] = jnp.zeros((L,), jnp.int32)
        vals_v[...] = lanes

        def bin_body(g, _):
            base = base_v[...]
            vals = vals_v[...]
            wxv = orig_v[0, :]
            hyv = orig_v[1, :]
            rminx = rects_v[0, pl.ds(g * L, L)]
            rmaxx = rects_v[1, pl.ds(g * L, L)]
            rminy = rects_v[2, pl.ds(g * L, L)]
            rmaxy = rects_v[3, pl.ds(g * L, L)]
            m1 = (jnp.minimum(rmaxx, wxv + (TILE - 1.0)) >
                  jnp.maximum(rminx, wxv))
            m2 = (jnp.minimum(rmaxy, hyv + (TILE - 1.0)) >
                  jnp.maximum(rminy, hyv))
            ones = jnp.full((L,), 1, jnp.int32)
            zeros = jnp.full((L,), 0, jnp.int32)
            mi = jnp.where(m1, ones, zeros) * jnp.where(m2, ones, zeros)
            m = mi > zeros
            pos = base + plsc.cumsum(mi) - mi
            plsc.store_scatter(idx_v, [pos], vals, mask=m)
            base_v[...] = base + plsc.all_reduce_population_count(m)
            vals_v[...] = vals + L
            return 0
        jax.lax.fori_loop(0, N // L, bin_body, 0)
        base = base_v[...]
        cnt = jnp.max(base)

        vals_v[...] = lanes

        def gat_body(g2, _):
            gidx = vals_v[...]
            members = idx_v[pl.ds(g2 * L, L)]
            members = jnp.minimum(jnp.maximum(members, jnp.full((L,), 0, jnp.int32)),
                                  jnp.full((L,), N - 1, jnp.int32))
            for a in range(9):
                asplat = jnp.full((L,), a, jnp.int32)
                vals = plsc.load_gather(prep_v, [asplat, members])
                plsc.store_scatter(pk_v, [asplat, gidx], vals)
            vals_v[...] = gidx + L
            return 0
        jax.lax.fori_loop(0, (cnt + L - 1) // L, gat_body, 0)

        nch = (cnt + K2 - 1) // K2
        for c in range(NCH):
            @pl.when(c < nch)
            def _():
                pltpu.sync_copy(pk_v.at[:, pl.ds(c * K2, K2)],
                                packed_hbm.at[t, c])
        cnt_v[...] = base
        pltpu.sync_copy(cnt_v, counts_hbm.at[t])


def _binning(rects, prep, origins):
    mesh = plsc.VectorSubcoreMesh(core_axis_name="c", subcore_axis_name="s")
    run = pl.kernel(
        _bin_kernel, mesh=mesh,
        compiler_params=pltpu.CompilerParams(needs_layout_passes=False),
        out_type=[
            jax.ShapeDtypeStruct((NT, NCH, 16, K2), jnp.float32),
            jax.ShapeDtypeStruct((NT, L), jnp.int32),
        ],
        scratch_types=[
            pltpu.VMEM((8, N), jnp.float32),
            pltpu.VMEM((16, N), jnp.float32),
            pltpu.VMEM((N,), jnp.int32),
            pltpu.VMEM((16, N), jnp.float32),
            pltpu.VMEM((L,), jnp.int32),
            pltpu.VMEM((2, L), jnp.float32),
            pltpu.VMEM((L,), jnp.int32),
            pltpu.VMEM((L,), jnp.int32),
        ],
    )
    return run(rects, prep, origins)


PAIR = 2


def _raster_kernel(cnt_ref, packed_ref, out_ref):
    i = pl.program_id(0)
    t0 = i * PAIR
    w0 = (t0 % NTX) * TILE
    h0 = (t0 // NTX) * TILE
    cnts = [cnt_ref[t0 + s] for s in range(PAIR)]
    nch = cnts[0]
    for s in range(1, PAIR):
        nch = jnp.maximum(nch, cnts[s])
    nch = (nch + K2 - 1) // K2

    pp = jax.lax.broadcasted_iota(jnp.int32, (1, P), 1)
    pyf = (h0 + pp // TILE).astype(jnp.float32)
    pxfs = [(w0 + s * TILE + pp % TILE).astype(jnp.float32)
            for s in range(PAIR)]

    tril = (jax.lax.broadcasted_iota(jnp.int32, (K2, K2), 1) <
            jax.lax.broadcasted_iota(jnp.int32, (K2, K2), 0)).astype(jnp.float32)
    gidx = jax.lax.broadcasted_iota(jnp.int32, (K2, 1), 0)

    def chunk_body(c, carry):
        out = []
        for s in range(PAIR):
            t_carry, acc_r, acc_g, acc_b, acc_a = carry[s]
            blk = jnp.transpose(packed_ref[s, c])
            mx = blk[:, 0:1]
            my = blk[:, 1:2]
            i00 = blk[:, 2:3]
            i11 = blk[:, 3:4]
            i01 = blk[:, 4:5]
            op = blk[:, 5:6]
            colr = blk[:, 6:7]
            colg = blk[:, 7:8]
            colb = blk[:, 8:9]
            valid = (gidx + c * K2) < cnts[s]
            dx = pxfs[s] - mx
            dy = pyf - my
            quad = dx * dx * i00 + dy * dy * i11 + 2.0 * (dx * dy) * i01
            gw = jnp.exp(-0.5 * quad)
            alpha = jnp.where(valid, jnp.minimum(gw * op, 0.99), 0.0)
            lg = jnp.log1p(-alpha)
            sexc = jax.lax.dot_general(
                tril, lg, (((1,), (0,)), ((), ())),
                precision=jax.lax.Precision.DEFAULT,
                preferred_element_type=jnp.float32)
            wgt = t_carry * jnp.exp(sexc) * alpha
            acc_r = acc_r + jnp.sum(wgt * colr, axis=0, keepdims=True)
            acc_g = acc_g + jnp.sum(wgt * colg, axis=0, keepdims=True)
            acc_b = acc_b + jnp.sum(wgt * colb, axis=0, keepdims=True)
            acc_a = acc_a + jnp.sum(wgt, axis=0, keepdims=True)
            t_carry = t_carry * jnp.exp(jnp.sum(lg, axis=0, keepdims=True))
            out.append((t_carry, acc_r, acc_g, acc_b, acc_a))
        return tuple(out)

    init1 = (jnp.ones((1, P), jnp.float32),) + \
            tuple(jnp.zeros((1, P), jnp.float32) for _ in range(4))
    fin = jax.lax.fori_loop(0, nch, chunk_body, (init1,) * PAIR)

    for s in range(PAIR):
        _, acc_r, acc_g, acc_b, acc_a = fin[s]
        resid = (1.0 - acc_a) * BKGD
        out_ref[s, 0:1, :] = acc_r + resid
        out_ref[s, 1:2, :] = acc_g + resid
        out_ref[s, 2:3, :] = acc_b + resid
        out_ref[s, 3:4, :] = acc_a


def _raster(cnt, packed):
    grid_spec = pltpu.PrefetchScalarGridSpec(
        num_scalar_prefetch=1,
        grid=(NT // PAIR,),
        in_specs=[
            pl.BlockSpec((PAIR, NCH, 16, K2), lambda t, cnt_ref: (t, 0, 0, 0)),
        ],
        out_specs=[
            pl.BlockSpec((PAIR, 8, P), lambda t, cnt_ref: (t, 0, 0)),
        ],
    )
    return pl.pallas_call(
        _raster_kernel,
        grid_spec=grid_spec,
        out_shape=[jax.ShapeDtypeStruct((NT, 8, P), jnp.float32)],
    )(cnt, packed)[0]


@jax.jit
def kernel(means2D, cov2d, color, opacity, depths):
    order = jnp.argsort(depths)
    attrs = jnp.stack([
        means2D[:, 0], means2D[:, 1],
        cov2d[:, 0, 0], cov2d[:, 0, 1], cov2d[:, 1, 1],
        opacity[:, 0],
        color[:, 0], color[:, 1], color[:, 2],
    ], axis=0)[:, order]
    attrs = jnp.concatenate(
        [attrs, jnp.zeros((16 - attrs.shape[0], N), jnp.float32)], axis=0)
    covu = jnp.stack([cov2d[:, 0, 0], cov2d[:, 0, 1], cov2d[:, 1, 1]], axis=0)
    covu = jnp.concatenate(
        [covu, jnp.zeros((8 - covu.shape[0], N), jnp.float32)], axis=0)

    prep, rects, rad = _prep(attrs, covu)
    tt = jnp.arange(NT, dtype=jnp.int32)
    origins = jnp.stack([
        jnp.broadcast_to(((tt % NTX) * TILE)[:, None], (NT, L)),
        jnp.broadcast_to(((tt // NTX) * TILE)[:, None], (NT, L)),
    ], axis=1).astype(jnp.float32)
    packed, counts = _binning(rects, prep, origins)
    cnt = counts[:, 0]
    out = _raster(cnt, packed)
    img = out[:, :4, :].reshape(NTY, NTX, 4, TILE, TILE)
    img = jnp.transpose(img, (0, 3, 1, 4, 2)).reshape(H, W, 4)
    return img[:, :, :3], img[:, :, 3:4], rad[0]


# speedup vs baseline: 4.8090x; 1.1008x over previous
import functools

import jax
import jax.numpy as jnp
from jax.experimental import pallas as pl
from jax.experimental.pallas import tpu as pltpu
from jax.experimental.pallas import tpu_sc as plsc

H = 128
W = 128
TILE = 16
N = 1024
NTX = W // TILE
NTY = H // TILE
NT = NTX * NTY
K2 = 128
NCH = N // K2
P = TILE * TILE
BKGD = 1.0


def _prep_kernel(attrs_ref, covu_ref, prep_ref, rects_ref, radii_ref):
    ca = covu_ref[0:1, :]
    cb = covu_ref[1:2, :]
    cd = covu_ref[2:3, :]
    det = ca * cd - cb * cb
    mid = 0.5 * (ca + cd)
    root = jnp.sqrt(jnp.maximum(mid * mid - det, 0.1))
    radii_ref[0:1, :] = jnp.ceil(3.0 * jnp.sqrt(mid + root))

    mx = attrs_ref[0:1, :]
    my = attrs_ref[1:2, :]
    ca = attrs_ref[2:3, :]
    cb = attrs_ref[3:4, :]
    cd = attrs_ref[4:5, :]
    det = ca * cd - cb * cb
    prep_ref[0:1, :] = mx
    prep_ref[1:2, :] = my
    prep_ref[2:3, :] = cd / det
    prep_ref[3:4, :] = ca / det
    prep_ref[4:5, :] = -cb / det
    prep_ref[5:16, :] = attrs_ref[5:16, :]

    mid = 0.5 * (ca + cd)
    root = jnp.sqrt(jnp.maximum(mid * mid - det, 0.1))
    rad = jnp.ceil(3.0 * jnp.sqrt(mid + root))
    rects_ref[0:1, :] = jnp.clip(mx - rad, 0.0, W - 1.0)
    rects_ref[1:2, :] = jnp.clip(mx + rad, 0.0, W - 1.0)
    rects_ref[2:3, :] = jnp.clip(my - rad, 0.0, H - 1.0)
    rects_ref[3:4, :] = jnp.clip(my + rad, 0.0, H - 1.0)
    rects_ref[4:8, :] = jnp.zeros((4, N), jnp.float32)


def _prep(attrs, covu):
    return pl.pallas_call(
        _prep_kernel,
        out_shape=[
            jax.ShapeDtypeStruct((16, N), jnp.float32),
            jax.ShapeDtypeStruct((8, N), jnp.float32),
            jax.ShapeDtypeStruct((1, N), jnp.float32),
        ],
    )(attrs, covu)


NC = 2
NS = 16
NWORK = NC * NS
L = 16


def _bin_kernel(rects_hbm, prep_hbm, orig_hbm, packed_hbm, counts_hbm,
                rects_v, prep_v, idx_v, pk_v, cnt_v, orig_v, base_v, vals_v):
    wid = jax.lax.axis_index("s") * NC + jax.lax.axis_index("c")
    pltpu.sync_copy(rects_hbm, rects_v)
    pltpu.sync_copy(prep_hbm, prep_v)
    lanes = jax.lax.iota(jnp.int32, L)

    for j in range(NT // NWORK):
        t = wid * (NT // NWORK) + j
        pltpu.sync_copy(orig_hbm.at[t], orig_v)
        base_v[...] = jnp.zeros((L,), jnp.int32)
        vals_v[...] = lanes

        def bin_body(g, _):
            base = base_v[...]
            vals = vals_v[...]
            wxv = orig_v[0, :]
            hyv = orig_v[1, :]
            rminx = rects_v[0, pl.ds(g * L, L)]
            rmaxx = rects_v[1, pl.ds(g * L, L)]
            rminy = rects_v[2, pl.ds(g * L, L)]
            rmaxy = rects_v[3, pl.ds(g * L, L)]
            m1 = (jnp.minimum(rmaxx, wxv + (TILE - 1.0)) >
                  jnp.maximum(rminx, wxv))
            m2 = (jnp.minimum(rmaxy, hyv + (TILE - 1.0)) >
                  jnp.maximum(rminy, hyv))
            ones = jnp.full((L,), 1, jnp.int32)
            zeros = jnp.full((L,), 0, jnp.int32)
            mi = jnp.where(m1, ones, zeros) * jnp.where(m2, ones, zeros)
            m = mi > zeros
            pos = base + plsc.cumsum(mi) - mi
            plsc.store_scatter(idx_v, [pos], vals, mask=m)
            base_v[...] = base + plsc.all_reduce_population_count(m)
            vals_v[...] = vals + L
            return 0
        jax.lax.fori_loop(0, N // L, bin_body, 0)
        base = base_v[...]
        cnt = jnp.max(base)

        vals_v[...] = lanes

        def gat_body(g2, _):
            gidx = vals_v[...]
            members = idx_v[pl.ds(g2 * L, L)]
            members = jnp.minimum(jnp.maximum(members, jnp.full((L,), 0, jnp.int32)),
                                  jnp.full((L,), N - 1, jnp.int32))
            for a in range(9):
                asplat = jnp.full((L,), a, jnp.int32)
                vals = plsc.load_gather(prep_v, [asplat, members])
                plsc.store_scatter(pk_v, [asplat, gidx], vals)
            vals_v[...] = gidx + L
            return 0
        jax.lax.fori_loop(0, (cnt + L - 1) // L, gat_body, 0)

        nch = (cnt + K2 - 1) // K2
        for c in range(NCH):
            @pl.when(c < nch)
            def _():
                pltpu.sync_copy(pk_v.at[:, pl.ds(c * K2, K2)],
                                packed_hbm.at[t, c])
        cnt_v[...] = base
        pltpu.sync_copy(cnt_v, counts_hbm.at[t])


def _binning(rects, prep, origins):
    mesh = plsc.VectorSubcoreMesh(core_axis_name="c", subcore_axis_name="s")
    run = pl.kernel(
        _bin_kernel, mesh=mesh,
        compiler_params=pltpu.CompilerParams(needs_layout_passes=False),
        out_type=[
            jax.ShapeDtypeStruct((NT, NCH, 16, K2), jnp.float32),
            jax.ShapeDtypeStruct((NT, L), jnp.int32),
        ],
        scratch_types=[
            pltpu.VMEM((8, N), jnp.float32),
            pltpu.VMEM((16, N), jnp.float32),
            pltpu.VMEM((N,), jnp.int32),
            pltpu.VMEM((16, N), jnp.float32),
            pltpu.VMEM((L,), jnp.int32),
            pltpu.VMEM((2, L), jnp.float32),
            pltpu.VMEM((L,), jnp.int32),
            pltpu.VMEM((L,), jnp.int32),
        ],
    )
    return run(rects, prep, origins)


PAIR = 4


def _raster_kernel(cnt_ref, packed_ref, out_ref):
    i = pl.program_id(0)
    t0 = i * PAIR
    w0 = (t0 % NTX) * TILE
    h0 = (t0 // NTX) * TILE
    cnts = [cnt_ref[t0 + s] for s in range(PAIR)]
    nch = cnts[0]
    for s in range(1, PAIR):
        nch = jnp.maximum(nch, cnts[s])
    nch = (nch + K2 - 1) // K2

    pp = jax.lax.broadcasted_iota(jnp.int32, (1, P), 1)
    pyf = (h0 + pp // TILE).astype(jnp.float32)
    pxfs = [(w0 + s * TILE + pp % TILE).astype(jnp.float32)
            for s in range(PAIR)]

    tril = (jax.lax.broadcasted_iota(jnp.int32, (K2, K2), 1) <
            jax.lax.broadcasted_iota(jnp.int32, (K2, K2), 0)).astype(jnp.float32)
    gidx = jax.lax.broadcasted_iota(jnp.int32, (K2, 1), 0)

    def chunk_body(c, carry):
        out = []
        for s in range(PAIR):
            t_carry, acc_r, acc_g, acc_b, acc_a = carry[s]
            blk = jnp.transpose(packed_ref[s, c])
            mx = blk[:, 0:1]
            my = blk[:, 1:2]
            i00 = blk[:, 2:3]
            i11 = blk[:, 3:4]
            i01 = blk[:, 4:5]
            op = blk[:, 5:6]
            colr = blk[:, 6:7]
            colg = blk[:, 7:8]
            colb = blk[:, 8:9]
            valid = (gidx + c * K2) < cnts[s]
            dx = pxfs[s] - mx
            dy = pyf - my
            quad = dx * dx * i00 + dy * dy * i11 + 2.0 * (dx * dy) * i01
            gw = jnp.exp(-0.5 * quad)
            alpha = jnp.where(valid, jnp.minimum(gw * op, 0.99), 0.0)
            lg = jnp.log1p(-alpha)
            sexc = jax.lax.dot_general(
                tril, lg, (((1,), (0,)), ((), ())),
                precision=jax.lax.Precision.DEFAULT,
                preferred_element_type=jnp.float32)
            wgt = t_carry * jnp.exp(sexc) * alpha
            acc_r = acc_r + jnp.sum(wgt * colr, axis=0, keepdims=True)
            acc_g = acc_g + jnp.sum(wgt * colg, axis=0, keepdims=True)
            acc_b = acc_b + jnp.sum(wgt * colb, axis=0, keepdims=True)
            acc_a = acc_a + jnp.sum(wgt, axis=0, keepdims=True)
            t_carry = t_carry * jnp.exp(jnp.sum(lg, axis=0, keepdims=True))
            out.append((t_carry, acc_r, acc_g, acc_b, acc_a))
        return tuple(out)

    init1 = (jnp.ones((1, P), jnp.float32),) + \
            tuple(jnp.zeros((1, P), jnp.float32) for _ in range(4))
    fin = jax.lax.fori_loop(0, nch, chunk_body, (init1,) * PAIR)

    for s in range(PAIR):
        _, acc_r, acc_g, acc_b, acc_a = fin[s]
        resid = (1.0 - acc_a) * BKGD
        out_ref[s, 0:1, :] = acc_r + resid
        out_ref[s, 1:2, :] = acc_g + resid
        out_ref[s, 2:3, :] = acc_b + resid
        out_ref[s, 3:4, :] = acc_a


def _raster(cnt, packed):
    grid_spec = pltpu.PrefetchScalarGridSpec(
        num_scalar_prefetch=1,
        grid=(NT // PAIR,),
        in_specs=[
            pl.BlockSpec((PAIR, NCH, 16, K2), lambda t, cnt_ref: (t, 0, 0, 0)),
        ],
        out_specs=[
            pl.BlockSpec((PAIR, 8, P), lambda t, cnt_ref: (t, 0, 0)),
        ],
    )
    return pl.pallas_call(
        _raster_kernel,
        grid_spec=grid_spec,
        out_shape=[jax.ShapeDtypeStruct((NT, 8, P), jnp.float32)],
    )(cnt, packed)[0]


@jax.jit
def kernel(means2D, cov2d, color, opacity, depths):
    order = jnp.argsort(depths)
    attrs = jnp.stack([
        means2D[:, 0], means2D[:, 1],
        cov2d[:, 0, 0], cov2d[:, 0, 1], cov2d[:, 1, 1],
        opacity[:, 0],
        color[:, 0], color[:, 1], color[:, 2],
    ], axis=0)[:, order]
    attrs = jnp.concatenate(
        [attrs, jnp.zeros((16 - attrs.shape[0], N), jnp.float32)], axis=0)
    covu = jnp.stack([cov2d[:, 0, 0], cov2d[:, 0, 1], cov2d[:, 1, 1]], axis=0)
    covu = jnp.concatenate(
        [covu, jnp.zeros((8 - covu.shape[0], N), jnp.float32)], axis=0)

    prep, rects, rad = _prep(attrs, covu)
    tt = jnp.arange(NT, dtype=jnp.int32)
    origins = jnp.stack([
        jnp.broadcast_to(((tt % NTX) * TILE)[:, None], (NT, L)),
        jnp.broadcast_to(((tt // NTX) * TILE)[:, None], (NT, L)),
    ], axis=1).astype(jnp.float32)
    packed, counts = _binning(rects, prep, origins)
    cnt = counts[:, 0]
    out = _raster(cnt, packed)
    img = out[:, :4, :].reshape(NTY, NTX, 4, TILE, TILE)
    img = jnp.transpose(img, (0, 3, 1, 4, 2)).reshape(H, W, 4)
    return img[:, :, :3], img[:, :, 3:4], rad[0]


# speedup vs baseline: 4.9424x; 1.0277x over previous
import functools

import jax
import jax.numpy as jnp
from jax.experimental import pallas as pl
from jax.experimental.pallas import tpu as pltpu
from jax.experimental.pallas import tpu_sc as plsc

H = 128
W = 128
TILE = 16
N = 1024
NTX = W // TILE
NTY = H // TILE
NT = NTX * NTY
K2 = 128
NCH = N // K2
P = TILE * TILE
BKGD = 1.0


def _prep_kernel(attrs_ref, covu_ref, prep_ref, rects_ref, radii_ref):
    ca = covu_ref[0:1, :]
    cb = covu_ref[1:2, :]
    cd = covu_ref[2:3, :]
    det = ca * cd - cb * cb
    mid = 0.5 * (ca + cd)
    root = jnp.sqrt(jnp.maximum(mid * mid - det, 0.1))
    radii_ref[0:1, :] = jnp.ceil(3.0 * jnp.sqrt(mid + root))

    mx = attrs_ref[0:1, :]
    my = attrs_ref[1:2, :]
    ca = attrs_ref[2:3, :]
    cb = attrs_ref[3:4, :]
    cd = attrs_ref[4:5, :]
    det = ca * cd - cb * cb
    prep_ref[0:1, :] = mx
    prep_ref[1:2, :] = my
    prep_ref[2:3, :] = cd / det
    prep_ref[3:4, :] = ca / det
    prep_ref[4:5, :] = -cb / det
    prep_ref[5:16, :] = attrs_ref[5:16, :]

    mid = 0.5 * (ca + cd)
    root = jnp.sqrt(jnp.maximum(mid * mid - det, 0.1))
    rad = jnp.ceil(3.0 * jnp.sqrt(mid + root))
    rects_ref[0:1, :] = jnp.clip(mx - rad, 0.0, W - 1.0)
    rects_ref[1:2, :] = jnp.clip(mx + rad, 0.0, W - 1.0)
    rects_ref[2:3, :] = jnp.clip(my - rad, 0.0, H - 1.0)
    rects_ref[3:4, :] = jnp.clip(my + rad, 0.0, H - 1.0)
    rects_ref[4:8, :] = jnp.zeros((4, N), jnp.float32)


def _prep(attrs, covu):
    return pl.pallas_call(
        _prep_kernel,
        out_shape=[
            jax.ShapeDtypeStruct((16, N), jnp.float32),
            jax.ShapeDtypeStruct((8, N), jnp.float32),
            jax.ShapeDtypeStruct((1, N), jnp.float32),
        ],
    )(attrs, covu)


NC = 2
NS = 16
NWORK = NC * NS
L = 16


def _bin_kernel(rects_hbm, prep_hbm, orig_hbm, packed_hbm, counts_hbm,
                rects_v, prep_v, idx_v, pk_v, cnt_v, orig_v, base_v, vals_v):
    wid = jax.lax.axis_index("s") * NC + jax.lax.axis_index("c")
    pltpu.sync_copy(rects_hbm, rects_v)
    pltpu.sync_copy(prep_hbm, prep_v)
    lanes = jax.lax.iota(jnp.int32, L)

    for j in range(NT // NWORK):
        t = wid * (NT // NWORK) + j
        pltpu.sync_copy(orig_hbm.at[t], orig_v)
        base_v[...] = jnp.zeros((L,), jnp.int32)
        vals_v[...] = lanes

        def bin_body(g, _):
            base = base_v[...]
            vals = vals_v[...]
            wxv = orig_v[0, :]
            hyv = orig_v[1, :]
            rminx = rects_v[0, pl.ds(g * L, L)]
            rmaxx = rects_v[1, pl.ds(g * L, L)]
            rminy = rects_v[2, pl.ds(g * L, L)]
            rmaxy = rects_v[3, pl.ds(g * L, L)]
            m1 = (jnp.minimum(rmaxx, wxv + (TILE - 1.0)) >
                  jnp.maximum(rminx, wxv))
            m2 = (jnp.minimum(rmaxy, hyv + (TILE - 1.0)) >
                  jnp.maximum(rminy, hyv))
            ones = jnp.full((L,), 1, jnp.int32)
            zeros = jnp.full((L,), 0, jnp.int32)
            mi = jnp.where(m1, ones, zeros) * jnp.where(m2, ones, zeros)
            m = mi > zeros
            pos = base + plsc.cumsum(mi) - mi
            plsc.store_scatter(idx_v, [pos], vals, mask=m)
            base_v[...] = base + plsc.all_reduce_population_count(m)
            vals_v[...] = vals + L
            return 0
        jax.lax.fori_loop(0, N // L, bin_body, 0)
        base = base_v[...]
        cnt = jnp.max(base)

        vals_v[...] = lanes

        def gat_body(g2, _):
            gidx = vals_v[...]
            members = idx_v[pl.ds(g2 * L, L)]
            members = jnp.minimum(jnp.maximum(members, jnp.full((L,), 0, jnp.int32)),
                                  jnp.full((L,), N - 1, jnp.int32))
            for a in range(9):
                asplat = jnp.full((L,), a, jnp.int32)
                vals = plsc.load_gather(prep_v, [asplat, members])
                plsc.store_scatter(pk_v, [asplat, gidx], vals)
            vals_v[...] = gidx + L
            return 0
        jax.lax.fori_loop(0, (cnt + L - 1) // L, gat_body, 0)

        nch = (cnt + K2 - 1) // K2
        for c in range(NCH):
            @pl.when(c < nch)
            def _():
                pltpu.sync_copy(pk_v.at[:, pl.ds(c * K2, K2)],
                                packed_hbm.at[t, c])
        cnt_v[...] = base
        pltpu.sync_copy(cnt_v, counts_hbm.at[t])


def _binning(rects, prep, origins):
    mesh = plsc.VectorSubcoreMesh(core_axis_name="c", subcore_axis_name="s")
    run = pl.kernel(
        _bin_kernel, mesh=mesh,
        compiler_params=pltpu.CompilerParams(needs_layout_passes=False),
        out_type=[
            jax.ShapeDtypeStruct((NT, NCH, 16, K2), jnp.float32),
            jax.ShapeDtypeStruct((NT, L), jnp.int32),
        ],
        scratch_types=[
            pltpu.VMEM((8, N), jnp.float32),
            pltpu.VMEM((16, N), jnp.float32),
            pltpu.VMEM((N,), jnp.int32),
            pltpu.VMEM((16, N), jnp.float32),
            pltpu.VMEM((L,), jnp.int32),
            pltpu.VMEM((2, L), jnp.float32),
            pltpu.VMEM((L,), jnp.int32),
            pltpu.VMEM((L,), jnp.int32),
        ],
    )
    return run(rects, prep, origins)


PAIR = 8


def _raster_kernel(cnt_ref, packed_ref, out_ref):
    i = pl.program_id(0)
    t0 = i * PAIR
    w0 = (t0 % NTX) * TILE
    h0 = (t0 // NTX) * TILE
    cnts = [cnt_ref[t0 + s] for s in range(PAIR)]
    nch = cnts[0]
    for s in range(1, PAIR):
        nch = jnp.maximum(nch, cnts[s])
    nch = (nch + K2 - 1) // K2

    pp = jax.lax.broadcasted_iota(jnp.int32, (1, P), 1)
    pyf = (h0 + pp // TILE).astype(jnp.float32)
    pxfs = [(w0 + s * TILE + pp % TILE).astype(jnp.float32)
            for s in range(PAIR)]

    tril = (jax.lax.broadcasted_iota(jnp.int32, (K2, K2), 1) <
            jax.lax.broadcasted_iota(jnp.int32, (K2, K2), 0)).astype(jnp.float32)
    gidx = jax.lax.broadcasted_iota(jnp.int32, (K2, 1), 0)

    def chunk_body(c, carry):
        out = []
        for s in range(PAIR):
            t_carry, acc_r, acc_g, acc_b, acc_a = carry[s]
            blk = jnp.transpose(packed_ref[s, c])
            mx = blk[:, 0:1]
            my = blk[:, 1:2]
            i00 = blk[:, 2:3]
            i11 = blk[:, 3:4]
            i01 = blk[:, 4:5]
            op = blk[:, 5:6]
            colr = blk[:, 6:7]
            colg = blk[:, 7:8]
            colb = blk[:, 8:9]
            valid = (gidx + c * K2) < cnts[s]
            dx = pxfs[s] - mx
            dy = pyf - my
            quad = dx * dx * i00 + dy * dy * i11 + 2.0 * (dx * dy) * i01
            gw = jnp.exp(-0.5 * quad)
            alpha = jnp.where(valid, jnp.minimum(gw * op, 0.99), 0.0)
            lg = jnp.log1p(-alpha)
            sexc = jax.lax.dot_general(
                tril, lg, (((1,), (0,)), ((), ())),
                precision=jax.lax.Precision.DEFAULT,
                preferred_element_type=jnp.float32)
            wgt = t_carry * jnp.exp(sexc) * alpha
            acc_r = acc_r + jnp.sum(wgt * colr, axis=0, keepdims=True)
            acc_g = acc_g + jnp.sum(wgt * colg, axis=0, keepdims=True)
            acc_b = acc_b + jnp.sum(wgt * colb, axis=0, keepdims=True)
            acc_a = acc_a + jnp.sum(wgt, axis=0, keepdims=True)
            t_carry = t_carry * jnp.exp(jnp.sum(lg, axis=0, keepdims=True))
            out.append((t_carry, acc_r, acc_g, acc_b, acc_a))
        return tuple(out)

    init1 = (jnp.ones((1, P), jnp.float32),) + \
            tuple(jnp.zeros((1, P), jnp.float32) for _ in range(4))
    fin = jax.lax.fori_loop(0, nch, chunk_body, (init1,) * PAIR)

    for s in range(PAIR):
        _, acc_r, acc_g, acc_b, acc_a = fin[s]
        resid = (1.0 - acc_a) * BKGD
        out_ref[s, 0:1, :] = acc_r + resid
        out_ref[s, 1:2, :] = acc_g + resid
        out_ref[s, 2:3, :] = acc_b + resid
        out_ref[s, 3:4, :] = acc_a


def _raster(cnt, packed):
    grid_spec = pltpu.PrefetchScalarGridSpec(
        num_scalar_prefetch=1,
        grid=(NT // PAIR,),
        in_specs=[
            pl.BlockSpec((PAIR, NCH, 16, K2), lambda t, cnt_ref: (t, 0, 0, 0)),
        ],
        out_specs=[
            pl.BlockSpec((PAIR, 8, P), lambda t, cnt_ref: (t, 0, 0)),
        ],
    )
    return pl.pallas_call(
        _raster_kernel,
        grid_spec=grid_spec,
        out_shape=[jax.ShapeDtypeStruct((NT, 8, P), jnp.float32)],
    )(cnt, packed)[0]


@jax.jit
def kernel(means2D, cov2d, color, opacity, depths):
    order = jnp.argsort(depths)
    attrs = jnp.stack([
        means2D[:, 0], means2D[:, 1],
        cov2d[:, 0, 0], cov2d[:, 0, 1], cov2d[:, 1, 1],
        opacity[:, 0],
        color[:, 0], color[:, 1], color[:, 2],
    ], axis=0)[:, order]
    attrs = jnp.concatenate(
        [attrs, jnp.zeros((16 - attrs.shape[0], N), jnp.float32)], axis=0)
    covu = jnp.stack([cov2d[:, 0, 0], cov2d[:, 0, 1], cov2d[:, 1, 1]], axis=0)
    covu = jnp.concatenate(
        [covu, jnp.zeros((8 - covu.shape[0], N), jnp.float32)], axis=0)

    prep, rects, rad = _prep(attrs, covu)
    tt = jnp.arange(NT, dtype=jnp.int32)
    origins = jnp.stack([
        jnp.broadcast_to(((tt % NTX) * TILE)[:, None], (NT, L)),
        jnp.broadcast_to(((tt // NTX) * TILE)[:, None], (NT, L)),
    ], axis=1).astype(jnp.float32)
    packed, counts = _binning(rects, prep, origins)
    cnt = counts[:, 0]
    out = _raster(cnt, packed)
    img = out[:, :4, :].reshape(NTY, NTX, 4, TILE, TILE)
    img = jnp.transpose(img, (0, 3, 1, 4, 2)).reshape(H, W, 4)
    return img[:, :, :3], img[:, :, 3:4], rad[0]
